# baseline (speedup 1.0000x reference)
import functools
import jax
import jax.numpy as jnp
from jax.experimental import pallas as pl
from jax.experimental.pallas import tpu as pltpu

_K = 10
_BIG = 3.0e38


def _knn_pool_body(lig_ref, posx_ref, posy_ref, posz_ref, x_ref,
                   w1_ref, b1_ref, w2_ref, b2_ref, out_ref, d2_ref,
                   vmask_ref, xbuf_ref, sem):
    npad = d2_ref.shape[1]
    bi = pl.program_id(0)
    nsc = 14
    sc = npad // nsc
    iota_c = jax.lax.broadcasted_iota(jnp.int32, (64, sc), 1)

    lx = lig_ref[0, :, 0:1]
    ly = lig_ref[0, :, 1:2]
    lz = lig_ref[0, :, 2:3]

    def acc_min(acc, rminc, idxc):
        minacc, idxacc = acc
        take = rminc < minacc
        return (jnp.where(take, rminc, minacc),
                jnp.where(take, idxc, idxacc))

    def rowmin(d2c, ii):
        rminc = jnp.min(d2c, axis=1, keepdims=True)
        cand = jnp.where(d2c == rminc, ii, npad)
        idxc = jnp.min(cand, axis=1, keepdims=True)
        return rminc, idxc

    acc0 = (jnp.full((64, 1), _BIG, jnp.float32),
            jnp.full((64, 1), npad, jnp.int32))

    def first_chunk(c, acc):
        s = c * sc
        dx = lx - posx_ref[0, 0:1, pl.ds(s, sc)]
        dy = ly - posy_ref[0, 0:1, pl.ds(s, sc)]
        dz = lz - posz_ref[0, 0:1, pl.ds(s, sc)]
        d2c = dx * dx + dy * dy + dz * dz
        d2_ref[:, pl.ds(s, sc)] = d2c
        vmask_ref[:, pl.ds(s, sc)] = jnp.zeros((1, sc), jnp.float32)
        return acc_min(acc, *rowmin(d2c, iota_c + s))

    _, idx = jax.lax.fori_loop(0, nsc, first_chunk, acc0)

    def kiter(_, prev_idx):
        def chunk_step(c, acc):
            s = c * sc
            ii = iota_c + s
            chosen = ii == prev_idx
            d2c = jnp.where(chosen, _BIG, d2_ref[:, pl.ds(s, sc)])
            d2_ref[:, pl.ds(s, sc)] = d2c
            hit = jnp.max(chosen.astype(jnp.float32), axis=0, keepdims=True)
            vmask_ref[:, pl.ds(s, sc)] = jnp.maximum(
                vmask_ref[:, pl.ds(s, sc)], hit)
            return acc_min(acc, *rowmin(d2c, ii))

        _, nidx = jax.lax.fori_loop(0, nsc, chunk_step, acc0)
        return nidx

    idx = jax.lax.fori_loop(0, _K - 1, kiter, idx)

    def final_chunk(c, cnt):
        s = c * sc
        chosen = (iota_c + s) == idx
        hit = jnp.max(chosen.astype(jnp.float32), axis=0, keepdims=True)
        m = jnp.maximum(vmask_ref[:, pl.ds(s, sc)], hit)
        vmask_ref[:, pl.ds(s, sc)] = m
        return cnt + jnp.sum(m)

    count = jax.lax.fori_loop(0, nsc, final_chunk, jnp.float32(0.0))

    nbuf, ch, d = xbuf_ref.shape
    nchunks = npad // ch

    def _copy(c, slot):
        return pltpu.make_async_copy(
            x_ref.at[bi, pl.ds(c * ch, ch), :], xbuf_ref.at[slot],
            sem.at[slot])

    _copy(0, 0).start()

    def pool_step(c, acc):
        slot = jax.lax.rem(c, nbuf)

        @pl.when(c + 1 < nchunks)
        def _():
            _copy(c + 1, jax.lax.rem(c + 1, nbuf)).start()

        mchunk = vmask_ref[:, pl.ds(c * ch, ch)].astype(jnp.bfloat16)
        _copy(c, slot).wait()
        return acc + jax.lax.dot_general(
            mchunk, xbuf_ref[slot], (((1,), (0,)), ((), ())),
            preferred_element_type=jnp.float32)

    pooled = jax.lax.fori_loop(
        0, nchunks, pool_step, jnp.zeros((1, d), dtype=jnp.float32))
    mean = pooled * (1.0 / count)

    h = jax.lax.dot_general(
        mean, w1_ref[...], (((1,), (1,)), ((), ())),
        preferred_element_type=jnp.float32) + b1_ref[...]
    h = h * jax.nn.sigmoid(h)
    out = jax.lax.dot_general(
        h, w2_ref[...], (((1,), (0,)), ((), ())),
        preferred_element_type=jnp.float32) + b2_ref[...]
    out_ref[0] = out


@functools.partial(jax.jit, static_argnames=())
def kernel(pos, x, lig_coords, W1, b1, gamma, beta, run_mean, run_var, W2, b2):
    B, N, D = x.shape
    L = lig_coords.shape[1]
    OUT = W2.shape[0]
    NPAD = 50176
    CH = NPAD // 8
    assert N <= NPAD

    posT = jnp.transpose(pos, (0, 2, 1))
    posT = jnp.pad(posT, ((0, 0), (0, 0), (0, NPAD - N)),
                   constant_values=1.0e4)
    posx = posT[:, 0:1, :]
    posy = posT[:, 1:2, :]
    posz = posT[:, 2:3, :]
    xp = jnp.pad(x, ((0, 0), (0, NPAD - N), (0, 0))).astype(jnp.bfloat16)

    scale = gamma * jax.lax.rsqrt(run_var + 1e-5)
    W1f = W1 * scale[:, None]
    b1f = ((b1 - run_mean) * scale + beta)[None, :]

    OPAD = ((OUT + 127) // 128) * 128
    W2T = jnp.pad(W2.T, ((0, 0), (0, OPAD - OUT)))
    b2p = jnp.pad(b2, (0, OPAD - OUT))[None, :]

    out = pl.pallas_call(
        _knn_pool_body,
        grid=(B,),
        in_specs=[
            pl.BlockSpec((1, L, 3), lambda b: (b, 0, 0)),
            pl.BlockSpec((1, 1, NPAD), lambda b: (b, 0, 0)),
            pl.BlockSpec((1, 1, NPAD), lambda b: (b, 0, 0)),
            pl.BlockSpec((1, 1, NPAD), lambda b: (b, 0, 0)),
            pl.BlockSpec(memory_space=pl.ANY),
            pl.BlockSpec((D, D), lambda b: (0, 0)),
            pl.BlockSpec((1, D), lambda b: (0, 0)),
            pl.BlockSpec((D, OPAD), lambda b: (0, 0)),
            pl.BlockSpec((1, OPAD), lambda b: (0, 0)),
        ],
        out_specs=pl.BlockSpec((1, 1, OPAD), lambda b: (b, 0, 0)),
        out_shape=jax.ShapeDtypeStruct((B, 1, OPAD), jnp.float32),
        scratch_shapes=[pltpu.VMEM((64, NPAD), jnp.float32),
                        pltpu.VMEM((1, NPAD), jnp.float32),
                        pltpu.VMEM((2, CH, D), jnp.bfloat16),
                        pltpu.SemaphoreType.DMA((2,))],
    )(lig_coords, posx, posy, posz, xp, W1f, b1f, W2T, b2p)
    return out[:, 0, :OUT]


# speedup vs baseline: 1.1683x; 1.1683x over previous
import functools
import jax
import jax.numpy as jnp
from jax import lax
from jax.experimental import pallas as pl
from jax.experimental.pallas import tpu as pltpu
from jax.experimental.pallas import tpu_sc as plsc

_K = 10
_BIG = 3.0e38
_NPAD = 50176
_NSC = 14
_DUMMY = _NPAD - 1


def _topk_body(lig_ref, posx_ref, posy_ref, posz_ref, idx_out_ref, d2_ref):
    npad = d2_ref.shape[1]
    sc = npad // _NSC
    iota_c = lax.broadcasted_iota(jnp.int32, (64, sc), 1)
    iota_k = lax.broadcasted_iota(jnp.int32, (64, 16), 1)

    lx = lig_ref[0, :, 0:1]
    ly = lig_ref[0, :, 1:2]
    lz = lig_ref[0, :, 2:3]

    def acc_min(acc, rminc, idxc):
        minacc, idxacc = acc
        take = rminc < minacc
        return (jnp.where(take, rminc, minacc),
                jnp.where(take, idxc, idxacc))

    def rowmin(d2c, ii):
        rminc = jnp.min(d2c, axis=1, keepdims=True)
        cand = jnp.where(d2c == rminc, ii, npad)
        idxc = jnp.min(cand, axis=1, keepdims=True)
        return rminc, idxc

    acc0 = (jnp.full((64, 1), _BIG, jnp.float32),
            jnp.full((64, 1), npad, jnp.int32))

    def first_chunk(c, acc):
        s = c * sc
        dx = lx - posx_ref[0, 0:1, pl.ds(s, sc)]
        dy = ly - posy_ref[0, 0:1, pl.ds(s, sc)]
        dz = lz - posz_ref[0, 0:1, pl.ds(s, sc)]
        d2c = dx * dx + dy * dy + dz * dz
        d2_ref[:, pl.ds(s, sc)] = d2c
        return acc_min(acc, *rowmin(d2c, iota_c + s))

    _, idx = lax.fori_loop(0, _NSC, first_chunk, acc0)
    picks = jnp.where(iota_k == 0, idx, _DUMMY)

    def kiter(k, carry):
        prev_idx, picks = carry

        def chunk_step(c, acc):
            s = c * sc
            ii = iota_c + s
            d2c = jnp.where(ii == prev_idx, _BIG, d2_ref[:, pl.ds(s, sc)])
            d2_ref[:, pl.ds(s, sc)] = d2c
            return acc_min(acc, *rowmin(d2c, ii))

        _, nidx = lax.fori_loop(0, _NSC, chunk_step, acc0)
        picks = jnp.where(iota_k == k + 1, nidx, picks)
        return nidx, picks

    _, picks = lax.fori_loop(0, _K - 1, kiter, (idx, picks))
    idx_out_ref[0] = picks


def _make_pool(B, D, npad):
    mesh = plsc.VectorSubcoreMesh(core_axis_name="c", subcore_axis_name="s")
    nchunks = 64

    @functools.partial(
        pl.kernel, mesh=mesh,
        out_type=jax.ShapeDtypeStruct((B, D + 16), jnp.float32),
        scratch_types=[
            pltpu.VMEM((npad,), jnp.int32),
            pltpu.VMEM((1024,), jnp.int32),
            pltpu.VMEM((1024,), jnp.int32),
            pltpu.VMEM((4, 16, D), jnp.float32),
            pltpu.VMEM((D + 16,), jnp.float32),
            pltpu.SemaphoreType.DMA((4,)),
        ],
        compiler_params=pltpu.CompilerParams(needs_layout_passes=False),
    )
    def pool(idx_hbm, x_hbm, out_hbm, marks, idxb, ieb, rows, stage, sem):
        wid = lax.axis_index("s") * 2 + lax.axis_index("c")

        @pl.when(wid < B)
        def _():
            b = wid
            pltpu.sync_copy(idx_hbm.at[b], idxb)
            iota16 = lax.iota(jnp.int32, 16)
            ones = jnp.full((16,), 1, jnp.int32)
            zeros = jnp.zeros((16,), jnp.int32)

            def mark(c, _):
                iv = idxb[pl.ds(c * 16, 16)]
                plsc.store_scatter(marks, [iv], ones)
                return 0

            lax.fori_loop(0, nchunks, mark, 0)

            def consume(c, cnt):
                base = c * 16
                iv = idxb[pl.ds(base, 16)]
                g = plsc.load_gather(marks, [iv])
                dup = jnp.zeros((16,), jnp.bool_)
                for s in range(1, 16):
                    shifted = plsc.load_gather(
                        idxb, [jnp.maximum(base + iota16 - s, 0)])
                    dup = jnp.logical_or(
                        dup, jnp.logical_and(iv == shifted, iota16 >= s))
                keep = jnp.logical_and(g == 1, jnp.logical_not(dup))
                plsc.store_scatter(marks, [iv], zeros)
                ie = jnp.where(keep, iv, _DUMMY) + b * npad
                ieb[pl.ds(base, 16)] = ie
                return cnt + plsc.all_reduce_population_count(keep)

            cnt = lax.fori_loop(0, nchunks, consume,
                                jnp.zeros((16,), jnp.int32))

            def fire(c, j):
                iv = ieb[pl.ds(c * 16, 16)]
                pltpu.make_async_copy(
                    x_hbm.at[iv], rows.at[j], sem.at[j]).start()

            def drain(c, j, acc):
                pltpu.make_async_copy(
                    x_hbm.at[ieb[pl.ds(c * 16, 16)]],
                    rows.at[j], sem.at[j]).wait()
                for r in range(16):
                    acc = tuple(
                        acc[v] + rows[j, r, pl.ds(v * 16, 16)]
                        for v in range(len(acc)))
                return acc

            def gather4(i, acc):
                c0 = i * 4
                for j in range(4):
                    fire(c0 + j, j)
                for j in range(4):
                    acc = drain(c0 + j, j, acc)
                return acc

            acc0 = tuple(jnp.zeros((16,), jnp.float32)
                         for _ in range(D // 16))
            acc = lax.fori_loop(0, nchunks // 4, gather4, acc0)

            for v in range(D // 16):
                stage[pl.ds(v * 16, 16)] = acc[v]
            stage[pl.ds(D, 16)] = cnt.astype(jnp.float32)
            pltpu.sync_copy(stage, out_hbm.at[b])

    return pool


def _mlp_body(p_ref, w1_ref, b1_ref, w2_ref, b2_ref, out_ref):
    d = w1_ref.shape[0]
    pooled = p_ref[:, :d]
    cnt = p_ref[:, d:d + 1] - 1.0
    mean = pooled * (1.0 / cnt)
    h = lax.dot_general(mean, w1_ref[...], (((1,), (1,)), ((), ())),
                        preferred_element_type=jnp.float32) + b1_ref[...]
    h = h * jax.nn.sigmoid(h)
    out_ref[...] = lax.dot_general(
        h, w2_ref[...], (((1,), (0,)), ((), ())),
        preferred_element_type=jnp.float32) + b2_ref[...]


@functools.partial(jax.jit, static_argnames=())
def kernel(pos, x, lig_coords, W1, b1, gamma, beta, run_mean, run_var, W2, b2):
    B, N, D = x.shape
    L = lig_coords.shape[1]
    OUT = W2.shape[0]
    NPAD = _NPAD
    assert N < NPAD

    posT = jnp.transpose(pos, (0, 2, 1))
    posT = jnp.pad(posT, ((0, 0), (0, 0), (0, NPAD - N)),
                   constant_values=1.0e4)
    xp = jnp.pad(x, ((0, 0), (0, NPAD - N), (0, 0)))

    scale = gamma * lax.rsqrt(run_var + 1e-5)
    W1f = W1 * scale[:, None]
    b1f = ((b1 - run_mean) * scale + beta)[None, :]
    OPAD = ((OUT + 127) // 128) * 128
    W2T = jnp.pad(W2.T, ((0, 0), (0, OPAD - OUT)))
    b2p = jnp.pad(b2, (0, OPAD - OUT))[None, :]

    picks = pl.pallas_call(
        _topk_body,
        grid=(B,),
        in_specs=[
            pl.BlockSpec((1, L, 3), lambda b: (b, 0, 0)),
            pl.BlockSpec((1, 1, NPAD), lambda b: (b, 0, 0)),
            pl.BlockSpec((1, 1, NPAD), lambda b: (b, 0, 0)),
            pl.BlockSpec((1, 1, NPAD), lambda b: (b, 0, 0)),
        ],
        out_specs=pl.BlockSpec((1, 64, 16), lambda b: (b, 0, 0)),
        out_shape=jax.ShapeDtypeStruct((B, 64, 16), jnp.int32),
        scratch_shapes=[pltpu.VMEM((64, NPAD), jnp.float32)],
    )(lig_coords, posT[:, 0:1, :], posT[:, 1:2, :], posT[:, 2:3, :])

    pooled = _make_pool(B, D, NPAD)(
        picks.reshape(B, 64 * 16), xp.reshape(B * NPAD, D))

    out = pl.pallas_call(
        _mlp_body,
        in_specs=[
            pl.BlockSpec((B, D + 16), lambda: (0, 0)),
            pl.BlockSpec((D, D), lambda: (0, 0)),
            pl.BlockSpec((1, D), lambda: (0, 0)),
            pl.BlockSpec((D, OPAD), lambda: (0, 0)),
            pl.BlockSpec((1, OPAD), lambda: (0, 0)),
        ],
        out_specs=pl.BlockSpec((B, OPAD), lambda: (0, 0)),
        out_shape=jax.ShapeDtypeStruct((B, OPAD), jnp.float32),
    )(pooled, W1f, b1f, W2T, b2p)
    return out[:, :OUT]


# speedup vs baseline: 2.1926x; 1.8768x over previous
import functools
import jax
import jax.numpy as jnp
from jax import lax
from jax.experimental import pallas as pl
from jax.experimental.pallas import tpu as pltpu
from jax.experimental.pallas import tpu_sc as plsc

_K = 10
_BIG = 3.0e38
_BIGI = 1 << 30
_NPAD = 50176
_NSC = 14
_BLK = 256
_NB = _NPAD // _BLK
_BMP = 224
_DUMMY = _NPAD - 1


def _scan_body(lig_ref, posx_ref, posy_ref, posz_ref, d2_ref, bm_ref):
    npad = d2_ref.shape[2]
    sc = npad // _NSC
    bpc = sc // _BLK

    lx = lig_ref[0, :, 0:1]
    ly = lig_ref[0, :, 1:2]
    lz = lig_ref[0, :, 2:3]

    bms = []
    for c in range(_NSC):
        s = c * sc
        dx = lx - posx_ref[0, 0:1, pl.ds(s, sc)]
        dy = ly - posy_ref[0, 0:1, pl.ds(s, sc)]
        dz = lz - posz_ref[0, 0:1, pl.ds(s, sc)]
        d2c = dx * dx + dy * dy + dz * dz
        d2_ref[0, :, pl.ds(s, sc)] = d2c
        for t in range(bpc):
            bms.append(jnp.min(
                d2c[:, t * _BLK:(t + 1) * _BLK], axis=1, keepdims=True))
    bms.append(jnp.full((64, _BMP - _NB), _BIG, jnp.float32))
    bm_ref[0] = jnp.concatenate(bms, axis=1)


def _make_topk(B, npad):
    mesh = plsc.VectorSubcoreMesh(core_axis_name="c", subcore_axis_name="s")
    nrows = 16

    @functools.partial(
        pl.kernel, mesh=mesh,
        out_type=jax.ShapeDtypeStruct((B * 64 * 16,), jnp.int32),
        scratch_types=[
            pltpu.VMEM((nrows * _BMP,), jnp.float32),
            pltpu.VMEM((nrows * 16,), jnp.int32),
            pltpu.VMEM((nrows * _BLK,), jnp.float32),
            pltpu.SemaphoreType.DMA,
        ],
        compiler_params=pltpu.CompilerParams(needs_layout_passes=False),
    )
    def topk(bm_hbm, d2_hbm, picks_hbm, bmb, knb, blkb, sem):
        wid = lax.axis_index("s") * 2 + lax.axis_index("c")
        iota16 = lax.iota(jnp.int32, 16)
        bigf = jnp.full((16,), _BIG, jnp.float32)

        pltpu.sync_copy(bm_hbm.at[pl.ds(wid * (nrows * _BMP), nrows * _BMP)],
                        bmb)
        for r in range(nrows):
            knb[pl.ds(r * 16, 16)] = jnp.full((16,), _DUMMY, jnp.int32)

        def pick_one(r, k):
            bv = bmb[pl.ds(r * _BMP, 16)]
            bid = iota16
            for j in range(1, _BMP // 16):
                v = bmb[pl.ds(r * _BMP + j * 16, 16)]
                take = v < bv
                bv = jnp.where(take, v, bv)
                bid = jnp.where(take, j * 16 + iota16, bid)
            m = jnp.min(bv)
            blk = jnp.min(jnp.where(bv == m, bid, _BIGI))
            base = blk * _BLK

            row = wid * nrows + r
            pltpu.sync_copy(
                d2_hbm.at[pl.ds(row * npad + base, _BLK)],
                blkb.at[pl.ds(r * _BLK, _BLK)])
            kn = knb[pl.ds(r * 16, 16)]
            rel = kn - base
            inr = jnp.logical_and(rel >= 0, rel < _BLK)
            rel = jnp.minimum(jnp.maximum(rel, 0), _BLK - 1)
            plsc.store_scatter(blkb, [r * _BLK + rel], bigf, mask=inr)

            cv = blkb[pl.ds(r * _BLK, 16)]
            ci = iota16
            for c in range(1, _BLK // 16):
                v = blkb[pl.ds(r * _BLK + c * 16, 16)]
                take = v < cv
                cv = jnp.where(take, v, cv)
                ci = jnp.where(take, c * 16 + iota16, ci)
            mv = jnp.min(cv)
            loc = jnp.min(jnp.where(cv == mv, ci, _BIGI))
            gidx = base + loc
            plsc.store_scatter(knb, [jnp.full((16,), r * 16, jnp.int32) + k],
                               jnp.full((16,), gidx, jnp.int32),
                               mask=iota16 == 0)

            plsc.store_scatter(blkb, [jnp.full((16,), r * _BLK, jnp.int32)
                                      + loc], bigf, mask=iota16 == 0)
            nv = blkb[pl.ds(r * _BLK, 16)]
            for c in range(1, _BLK // 16):
                nv = jnp.minimum(nv, blkb[pl.ds(r * _BLK + c * 16, 16)])
            plsc.store_scatter(bmb, [jnp.full((16,), r * _BMP, jnp.int32)
                                     + blk],
                               jnp.full((16,), jnp.min(nv), jnp.float32),
                               mask=iota16 == 0)
            return k

        for k in range(_K):
            lax.fori_loop(0, nrows, pick_one, k)

        pltpu.sync_copy(knb, picks_hbm.at[pl.ds(wid * (nrows * 16),
                                                nrows * 16)])

    return topk


def _make_pool(B, D, npad):
    mesh = plsc.VectorSubcoreMesh(core_axis_name="c", subcore_axis_name="s")
    nchunks = 64

    @functools.partial(
        pl.kernel, mesh=mesh,
        out_type=jax.ShapeDtypeStruct((B, D + 16), jnp.float32),
        scratch_types=[
            pltpu.VMEM((npad,), jnp.int32),
            pltpu.VMEM((1024,), jnp.int32),
            pltpu.VMEM((1024,), jnp.int32),
            pltpu.VMEM((4, 16, D), jnp.float32),
            pltpu.VMEM((D + 16,), jnp.float32),
            pltpu.SemaphoreType.DMA((4,)),
        ],
        compiler_params=pltpu.CompilerParams(needs_layout_passes=False),
    )
    def pool(idx_hbm, x_hbm, out_hbm, marks, idxb, ieb, rows, stage, sem):
        wid = lax.axis_index("s") * 2 + lax.axis_index("c")

        @pl.when(wid < B)
        def _():
            b = wid
            pltpu.sync_copy(idx_hbm.at[b], idxb)
            iota16 = lax.iota(jnp.int32, 16)
            ones = jnp.full((16,), 1, jnp.int32)
            zeros = jnp.zeros((16,), jnp.int32)

            def mark(c, _):
                iv = idxb[pl.ds(c * 16, 16)]
                plsc.store_scatter(marks, [iv], ones)
                return 0

            lax.fori_loop(0, nchunks, mark, 0)

            def consume(c, cnt):
                base = c * 16
                iv = idxb[pl.ds(base, 16)]
                g = plsc.load_gather(marks, [iv])
                dup = jnp.zeros((16,), jnp.bool_)
                for s in range(1, 16):
                    shifted = plsc.load_gather(
                        idxb, [jnp.maximum(base + iota16 - s, 0)])
                    dup = jnp.logical_or(
                        dup, jnp.logical_and(iv == shifted, iota16 >= s))
                keep = jnp.logical_and(g == 1, jnp.logical_not(dup))
                plsc.store_scatter(marks, [iv], zeros)
                ie = jnp.where(keep, iv, _DUMMY) + b * npad
                ieb[pl.ds(base, 16)] = ie
                return cnt + plsc.all_reduce_population_count(keep)

            cnt = lax.fori_loop(0, nchunks, consume,
                                jnp.zeros((16,), jnp.int32))

            def fire(c, j):
                iv = ieb[pl.ds(c * 16, 16)]
                pltpu.make_async_copy(
                    x_hbm.at[iv], rows.at[j], sem.at[j]).start()

            def drain(c, j, acc):
                pltpu.make_async_copy(
                    x_hbm.at[ieb[pl.ds(c * 16, 16)]],
                    rows.at[j], sem.at[j]).wait()
                for r in range(16):
                    acc = tuple(
                        acc[v] + rows[j, r, pl.ds(v * 16, 16)]
                        for v in range(len(acc)))
                return acc

            def gather4(i, acc):
                c0 = i * 4
                for j in range(4):
                    fire(c0 + j, j)
                for j in range(4):
                    acc = drain(c0 + j, j, acc)
                return acc

            acc0 = tuple(jnp.zeros((16,), jnp.float32)
                         for _ in range(D // 16))
            acc = lax.fori_loop(0, nchunks // 4, gather4, acc0)

            for v in range(D // 16):
                stage[pl.ds(v * 16, 16)] = acc[v]
            stage[pl.ds(D, 16)] = cnt.astype(jnp.float32)
            pltpu.sync_copy(stage, out_hbm.at[b])

    return pool


def _mlp_body(p_ref, w1_ref, b1_ref, w2_ref, b2_ref, out_ref):
    d = w1_ref.shape[0]
    pooled = p_ref[:, :d]
    cnt = p_ref[:, d:d + 1] - 1.0
    mean = pooled * (1.0 / cnt)
    h = lax.dot_general(mean, w1_ref[...], (((1,), (1,)), ((), ())),
                        preferred_element_type=jnp.float32) + b1_ref[...]
    h = h * jax.nn.sigmoid(h)
    out_ref[...] = lax.dot_general(
        h, w2_ref[...], (((1,), (0,)), ((), ())),
        preferred_element_type=jnp.float32) + b2_ref[...]


@functools.partial(jax.jit, static_argnames=())
def kernel(pos, x, lig_coords, W1, b1, gamma, beta, run_mean, run_var, W2, b2):
    B, N, D = x.shape
    L = lig_coords.shape[1]
    OUT = W2.shape[0]
    NPAD = _NPAD
    assert N < NPAD

    posT = jnp.transpose(pos, (0, 2, 1))
    posT = jnp.pad(posT, ((0, 0), (0, 0), (0, NPAD - N)),
                   constant_values=1.0e4)
    xp = jnp.pad(x, ((0, 0), (0, NPAD - N), (0, 0)))

    scale = gamma * lax.rsqrt(run_var + 1e-5)
    W1f = W1 * scale[:, None]
    b1f = ((b1 - run_mean) * scale + beta)[None, :]
    OPAD = ((OUT + 127) // 128) * 128
    W2T = jnp.pad(W2.T, ((0, 0), (0, OPAD - OUT)))
    b2p = jnp.pad(b2, (0, OPAD - OUT))[None, :]

    d2, bm = pl.pallas_call(
        _scan_body,
        grid=(B,),
        in_specs=[
            pl.BlockSpec((1, L, 3), lambda b: (b, 0, 0)),
            pl.BlockSpec((1, 1, NPAD), lambda b: (b, 0, 0)),
            pl.BlockSpec((1, 1, NPAD), lambda b: (b, 0, 0)),
            pl.BlockSpec((1, 1, NPAD), lambda b: (b, 0, 0)),
        ],
        out_specs=[pl.BlockSpec((1, 64, NPAD), lambda b: (b, 0, 0)),
                   pl.BlockSpec((1, 64, _BMP), lambda b: (b, 0, 0))],
        out_shape=[jax.ShapeDtypeStruct((B, 64, NPAD), jnp.float32),
                   jax.ShapeDtypeStruct((B, 64, _BMP), jnp.float32)],
    )(lig_coords, posT[:, 0:1, :], posT[:, 1:2, :], posT[:, 2:3, :])

    picks = _make_topk(B, NPAD)(
        bm.reshape(B * 64 * _BMP), d2.reshape(B * 64 * NPAD))

    pooled = _make_pool(B, D, NPAD)(
        picks.reshape(B, 64 * 16), xp.reshape(B * NPAD, D))

    out = pl.pallas_call(
        _mlp_body,
        in_specs=[
            pl.BlockSpec((B, D + 16), lambda: (0, 0)),
            pl.BlockSpec((D, D), lambda: (0, 0)),
            pl.BlockSpec((1, D), lambda: (0, 0)),
            pl.BlockSpec((D, OPAD), lambda: (0, 0)),
            pl.BlockSpec((1, OPAD), lambda: (0, 0)),
        ],
        out_specs=pl.BlockSpec((B, OPAD), lambda: (0, 0)),
        out_shape=jax.ShapeDtypeStruct((B, OPAD), jnp.float32),
    )(pooled, W1f, b1f, W2T, b2p)
    return out[:, :OUT]


# speedup vs baseline: 3.2771x; 1.4946x over previous
import functools
import jax
import jax.numpy as jnp
from jax import lax
from jax.experimental import pallas as pl
from jax.experimental.pallas import tpu as pltpu
from jax.experimental.pallas import tpu_sc as plsc

_K = 10
_BIG = 3.0e38
_BIGI = 1 << 30
_NPAD = 50176
_NSC = 14
_BLK = 256
_NB = _NPAD // _BLK
_BMP = 224
_DUMMY = _NPAD - 1


def _scan_body(lig_ref, posx_ref, posy_ref, posz_ref, d2_ref, bm_ref):
    npad = d2_ref.shape[1]
    sc = npad // _NSC
    bpc = sc // _BLK

    lx = lig_ref[0, :, 0:1]
    ly = lig_ref[0, :, 1:2]
    lz = lig_ref[0, :, 2:3]

    bms = []
    for c in range(_NSC):
        s = c * sc
        dx = lx - posx_ref[0, 0:1, pl.ds(s, sc)]
        dy = ly - posy_ref[0, 0:1, pl.ds(s, sc)]
        dz = lz - posz_ref[0, 0:1, pl.ds(s, sc)]
        d2c = dx * dx + dy * dy + dz * dz
        d2_ref[:, pl.ds(s, sc)] = d2c
        for t in range(bpc):
            bms.append(jnp.min(
                d2c[:, t * _BLK:(t + 1) * _BLK], axis=1, keepdims=True))
    bms.append(jnp.full((64, _BMP - _NB), _BIG, jnp.float32))
    bm_ref[...] = jnp.concatenate(bms, axis=1)


def _make_topk(B, npad):
    mesh = plsc.VectorSubcoreMesh(core_axis_name="c", subcore_axis_name="s")
    nrows = 16

    @functools.partial(
        pl.kernel, mesh=mesh,
        out_type=jax.ShapeDtypeStruct((B * 64, 16), jnp.int32),
        scratch_types=[
            pltpu.VMEM((nrows * _BMP,), jnp.float32),
            pltpu.VMEM((nrows, 16), jnp.int32),
            pltpu.VMEM((nrows * _BLK,), jnp.float32),
            pltpu.VMEM((16,), jnp.int32),
            pltpu.SemaphoreType.DMA,
        ],
        compiler_params=pltpu.CompilerParams(needs_layout_passes=False),
    )
    def topk(bm_hbm, d2_hbm, picks_hbm, bmb, knb, blkb, blkv, sem):
        wid = lax.axis_index("s") * 2 + lax.axis_index("c")
        iota16 = lax.iota(jnp.int32, 16)
        bigf = jnp.full((16,), _BIG, jnp.float32)

        pltpu.sync_copy(bm_hbm.at[pl.ds(wid * (nrows * _BMP), nrows * _BMP)],
                        bmb)
        for r in range(nrows):
            knb[r, :] = jnp.full((16,), _DUMMY, jnp.int32)

        def fire_one(r, k):
            bv = bmb[pl.ds(r * _BMP, 16)]
            bid = iota16
            for j in range(1, _BMP // 16):
                v = bmb[pl.ds(r * _BMP + j * 16, 16)]
                take = v < bv
                bv = jnp.where(take, v, bv)
                bid = jnp.where(take, j * 16 + iota16, bid)
            m = jnp.min(bv)
            blk = jnp.min(jnp.where(bv == m, bid, _BIGI))
            plsc.store_scatter(blkv, [jnp.full((16,), r, jnp.int32)],
                               jnp.full((16,), blk, jnp.int32),
                               mask=iota16 == 0)
            row = wid * nrows + r
            pltpu.make_async_copy(
                d2_hbm.at[row, pl.ds(blk * _BLK, _BLK)],
                blkb.at[pl.ds(r * _BLK, _BLK)], sem).start()
            return k

        def wait_one(r, k):
            pltpu.make_async_copy(
                d2_hbm.at[0, pl.ds(0, _BLK)],
                blkb.at[pl.ds(r * _BLK, _BLK)], sem).wait()
            return k

        def proc_one(r, k):
            blk = jnp.min(jnp.where(iota16 == r, blkv[0:16], _BIGI))
            base = blk * _BLK

            kn = knb[r, :]
            rel = kn - base
            inr = jnp.logical_and(rel >= 0, rel < _BLK)
            rel = jnp.minimum(jnp.maximum(rel, 0), _BLK - 1)
            plsc.store_scatter(blkb, [r * _BLK + rel], bigf, mask=inr)

            cv = blkb[pl.ds(r * _BLK, 16)]
            ci = iota16
            for c in range(1, _BLK // 16):
                v = blkb[pl.ds(r * _BLK + c * 16, 16)]
                take = v < cv
                cv = jnp.where(take, v, cv)
                ci = jnp.where(take, c * 16 + iota16, ci)
            mv = jnp.min(cv)
            loc = jnp.min(jnp.where(cv == mv, ci, _BIGI))
            gidx = base + loc
            plsc.store_scatter(knb, [jnp.full((16,), r, jnp.int32),
                                     jnp.full((16,), k, jnp.int32)],
                               jnp.full((16,), gidx, jnp.int32),
                               mask=iota16 == 0)

            plsc.store_scatter(blkb, [jnp.full((16,), r * _BLK, jnp.int32)
                                      + loc], bigf, mask=iota16 == 0)
            nv = blkb[pl.ds(r * _BLK, 16)]
            for c in range(1, _BLK // 16):
                nv = jnp.minimum(nv, blkb[pl.ds(r * _BLK + c * 16, 16)])
            plsc.store_scatter(bmb, [jnp.full((16,), r * _BMP, jnp.int32)
                                     + blk],
                               jnp.full((16,), jnp.min(nv), jnp.float32),
                               mask=iota16 == 0)
            return k

        for k in range(_K):
            lax.fori_loop(0, nrows, fire_one, k)
            lax.fori_loop(0, nrows, wait_one, k)
            lax.fori_loop(0, nrows, proc_one, k)

        pltpu.sync_copy(knb, picks_hbm.at[pl.ds(wid * nrows, nrows), :])

    return topk


def _make_pool(B, D, npad):
    mesh = plsc.VectorSubcoreMesh(core_axis_name="c", subcore_axis_name="s")
    nchunks = 64

    @functools.partial(
        pl.kernel, mesh=mesh,
        out_type=jax.ShapeDtypeStruct((B, D + 16), jnp.float32),
        scratch_types=[
            pltpu.VMEM((npad,), jnp.int32),
            pltpu.VMEM((1024,), jnp.int32),
            pltpu.VMEM((1024,), jnp.int32),
            pltpu.VMEM((8, 16, D), jnp.float32),
            pltpu.VMEM((D + 16,), jnp.float32),
            pltpu.SemaphoreType.DMA((8,)),
        ],
        compiler_params=pltpu.CompilerParams(needs_layout_passes=False),
    )
    def pool(idx_hbm, x_hbm, out_hbm, marks, idxb, ieb, rows, stage, sem):
        wid = lax.axis_index("s") * 2 + lax.axis_index("c")

        @pl.when(wid < B)
        def _():
            b = wid
            pltpu.sync_copy(idx_hbm.at[b], idxb)
            iota16 = lax.iota(jnp.int32, 16)
            ones = jnp.full((16,), 1, jnp.int32)
            zeros = jnp.zeros((16,), jnp.int32)

            def mark(c, _):
                iv = idxb[pl.ds(c * 16, 16)]
                plsc.store_scatter(marks, [iv], ones)
                return 0

            lax.fori_loop(0, nchunks, mark, 0)

            def consume(c, cnt):
                base = c * 16
                iv = idxb[pl.ds(base, 16)]
                g = plsc.load_gather(marks, [iv])
                dup = jnp.zeros((16,), jnp.bool_)
                for s in range(1, 16):
                    shifted = plsc.load_gather(
                        idxb, [jnp.maximum(base + iota16 - s, 0)])
                    dup = jnp.logical_or(
                        dup, jnp.logical_and(iv == shifted, iota16 >= s))
                keep = jnp.logical_and(g == 1, jnp.logical_not(dup))
                plsc.store_scatter(marks, [iv], zeros)
                ie = jnp.where(keep, iv, _DUMMY) + b * npad
                ieb[pl.ds(base, 16)] = ie
                return cnt + plsc.all_reduce_population_count(keep)

            cnt = lax.fori_loop(0, nchunks, consume,
                                jnp.zeros((16,), jnp.int32))

            def fire(c, j):
                iv = ieb[pl.ds(c * 16, 16)]
                pltpu.make_async_copy(
                    x_hbm.at[iv], rows.at[j], sem.at[j]).start()

            def drain(c, j, acc):
                pltpu.make_async_copy(
                    x_hbm.at[ieb[pl.ds(c * 16, 16)]],
                    rows.at[j], sem.at[j]).wait()
                for r in range(16):
                    acc = tuple(
                        acc[v] + rows[j, r, pl.ds(v * 16, 16)]
                        for v in range(len(acc)))
                return acc

            def gather8(i, acc):
                c0 = i * 8
                for j in range(8):
                    fire(c0 + j, j)
                for j in range(8):
                    acc = drain(c0 + j, j, acc)
                return acc

            acc0 = tuple(jnp.zeros((16,), jnp.float32)
                         for _ in range(D // 16))
            acc = lax.fori_loop(0, nchunks // 8, gather8, acc0)

            for v in range(D // 16):
                stage[pl.ds(v * 16, 16)] = acc[v]
            stage[pl.ds(D, 16)] = cnt.astype(jnp.float32)
            pltpu.sync_copy(stage, out_hbm.at[b])

    return pool


def _mlp_body(p_ref, w1_ref, b1_ref, w2_ref, b2_ref, out_ref):
    d = w1_ref.shape[0]
    pooled = p_ref[:, :d]
    cnt = p_ref[:, d:d + 1] - 1.0
    mean = pooled * (1.0 / cnt)
    h = lax.dot_general(mean, w1_ref[...], (((1,), (1,)), ((), ())),
                        preferred_element_type=jnp.float32) + b1_ref[...]
    h = h * jax.nn.sigmoid(h)
    out_ref[...] = lax.dot_general(
        h, w2_ref[...], (((1,), (0,)), ((), ())),
        preferred_element_type=jnp.float32) + b2_ref[...]


@functools.partial(jax.jit, static_argnames=())
def kernel(pos, x, lig_coords, W1, b1, gamma, beta, run_mean, run_var, W2, b2):
    B, N, D = x.shape
    L = lig_coords.shape[1]
    OUT = W2.shape[0]
    NPAD = _NPAD
    assert N < NPAD

    posT = jnp.transpose(pos, (0, 2, 1))
    posT = jnp.pad(posT, ((0, 0), (0, 0), (0, NPAD - N)),
                   constant_values=1.0e4)
    xp = jnp.pad(x, ((0, 0), (0, NPAD - N), (0, 0)))

    scale = gamma * lax.rsqrt(run_var + 1e-5)
    W1f = W1 * scale[:, None]
    b1f = ((b1 - run_mean) * scale + beta)[None, :]
    OPAD = ((OUT + 127) // 128) * 128
    W2T = jnp.pad(W2.T, ((0, 0), (0, OPAD - OUT)))
    b2p = jnp.pad(b2, (0, OPAD - OUT))[None, :]

    d2, bm = pl.pallas_call(
        _scan_body,
        grid=(B,),
        in_specs=[
            pl.BlockSpec((1, L, 3), lambda b: (b, 0, 0)),
            pl.BlockSpec((1, 1, NPAD), lambda b: (b, 0, 0)),
            pl.BlockSpec((1, 1, NPAD), lambda b: (b, 0, 0)),
            pl.BlockSpec((1, 1, NPAD), lambda b: (b, 0, 0)),
        ],
        out_specs=[pl.BlockSpec((64, NPAD), lambda b: (b, 0)),
                   pl.BlockSpec((64, _BMP), lambda b: (b, 0))],
        out_shape=[jax.ShapeDtypeStruct((B * 64, NPAD), jnp.float32),
                   jax.ShapeDtypeStruct((B * 64, _BMP), jnp.float32)],
    )(lig_coords, posT[:, 0:1, :], posT[:, 1:2, :], posT[:, 2:3, :])

    picks = _make_topk(B, NPAD)(bm.reshape(B * 64 * _BMP), d2)

    pooled = _make_pool(B, D, NPAD)(
        picks.reshape(B, 64 * 16), xp.reshape(B * NPAD, D))

    out = pl.pallas_call(
        _mlp_body,
        in_specs=[
            pl.BlockSpec((B, D + 16), lambda: (0, 0)),
            pl.BlockSpec((D, D), lambda: (0, 0)),
            pl.BlockSpec((1, D), lambda: (0, 0)),
            pl.BlockSpec((D, OPAD), lambda: (0, 0)),
            pl.BlockSpec((1, OPAD), lambda: (0, 0)),
        ],
        out_specs=pl.BlockSpec((B, OPAD), lambda: (0, 0)),
        out_shape=jax.ShapeDtypeStruct((B, OPAD), jnp.float32),
    )(pooled, W1f, b1f, W2T, b2p)
    return out[:, :OUT]
